# baseline (speedup 1.0000x reference)
import functools

import jax
import jax.numpy as jnp
from jax import lax
from jax.experimental import pallas as pl
from jax.experimental.pallas import tpu as pltpu
from jax.experimental.pallas import tpu_sc as plsc

N = 10000
E = 32000
D = 2560
H = 8
C = 320
G = 16
HC = H * C
CW = 128
NCH = HC // CW
BN = 1024
BC = 512

NSC = 2
NSUB = 16
NW = NSC * NSUB
EPT = 1024
EPAD = NW * EPT
SB = 128
NSB = EPT // SB
NPT = 624
NPT_LAST = N - NPT * (NSUB - 1)


def _mm_body(x_ref, w_ref, b_ref, o_ref):
    acc = jnp.dot(x_ref[...], w_ref[...], preferred_element_type=jnp.float32)
    acc = acc + b_ref[0:1, :]
    for i in range(BC // CW):
        o_ref[i] = acc[:, i * CW:(i + 1) * CW]


def _proj_chunkmajor(x, Wcat, bcat2d):
    n_blocks = pl.cdiv(N, BN)
    c_blocks = (2 * HC) // BC
    return pl.pallas_call(
        _mm_body,
        grid=(n_blocks, c_blocks),
        in_specs=[
            pl.BlockSpec((BN, D), lambda n, c: (n, 0)),
            pl.BlockSpec((D, BC), lambda n, c: (0, c)),
            pl.BlockSpec((8, BC), lambda n, c: (0, c)),
        ],
        out_specs=pl.BlockSpec((BC // CW, BN, CW), lambda n, c: (c, n, 0)),
        out_shape=jax.ShapeDtypeStruct((2 * NCH, N, CW), jnp.float32),
    )(x, Wcat, bcat2d)


def _head_of(ch, j):
    return (ch * CW + j * 16) // C


def _onehot(h):
    return jnp.where(lax.iota(jnp.int32, 16) == h, 1.0, 0.0)


def _hsum(v):
    i = lax.iota(jnp.int32, 16)
    for k in (1, 2, 4, 8):
        v = v + v.at[jnp.bitwise_xor(i, k)].get(mode="promise_in_bounds")
    return v[0]


def _sc_logits_body(xl_f, xr_f, src_h, dst_h, ea_h, wea_h, att_h,
                    ex_out, den_out,
                    src_v, dst_v, ea_v, lac, exb,
                    rows_l, rows_r, idx_l, idx_r, dst_idx,
                    wea_v, att_v, den_sh, sem):
    cid = lax.axis_index("c")
    sid = lax.axis_index("s")
    wid = cid * NSUB + sid
    base = wid * EPT

    pltpu.sync_copy(src_h.at[pl.ds(base, EPT)], src_v)
    pltpu.sync_copy(dst_h.at[pl.ds(base, EPT)], dst_v)
    pltpu.sync_copy(ea_h.at[pl.ds(base, EPT)], ea_v.at[pl.ds(0, EPT)])
    pltpu.sync_copy(wea_h, wea_v)
    pltpu.sync_copy(att_h, att_v)

    def _zero(i, _):
        lac[i] = jnp.zeros((16,), jnp.float32)
        return 0
    lax.fori_loop(0, EPT, _zero, 0)
    pltpu.sync_copy(lac.at[pl.ds(0, NPT)], den_sh.at[pl.ds(sid * NPT, NPT)])

    @pl.when(sid == NSUB - 1)
    def _zero_tail():
        pltpu.sync_copy(lac.at[pl.ds(0, NPT_LAST - NPT)],
                        den_sh.at[pl.ds(NPT * NSUB, NPT_LAST - NPT)])
    plsc.subcore_barrier()

    for ch in range(NCH):
        wv = [wea_v[ch, pl.ds(j * 16, 16)] for j in range(8)]
        av = [att_v[ch, pl.ds(j * 16, 16)] for j in range(8)]

        def _sb_body(sb, _, ch=ch, wv=wv, av=av):
            eb = sb * SB
            for j in range(SB // 16):
                s16 = src_v[pl.ds(eb + j * 16, 16)]
                d16 = dst_v[pl.ds(eb + j * 16, 16)]
                idx_l[pl.ds(j * 16, 16)] = s16 + ch * N
                idx_r[pl.ds(j * 16, 16)] = d16 + ch * N
            pltpu.async_copy(xl_f.at[idx_l], rows_l, sem).wait()
            pltpu.async_copy(xr_f.at[idx_r], rows_r, sem).wait()

            def _edge(e, _):
                ea_s = ea_v[pl.ds(eb + e, 16)][0]
                acc = [jnp.zeros((16,), jnp.float32),
                       jnp.zeros((16,), jnp.float32)]
                h0 = _head_of(ch, 0)
                for j in range(8):
                    z = (rows_l[e, pl.ds(j * 16, 16)]
                         + rows_r[e, pl.ds(j * 16, 16)]
                         + ea_s * wv[j])
                    lr = jnp.maximum(z, 0.0) + 0.2 * jnp.minimum(z, 0.0)
                    g = _head_of(ch, j) - h0
                    acc[g] = acc[g] + lr * av[j]
                h1 = _head_of(ch, 7)
                upd = _hsum(acc[0]) * _onehot(h0)
                if h1 != h0:
                    upd = upd + _hsum(acc[1]) * _onehot(h1)
                lac[eb + e] = lac[eb + e] + upd
                return 0
            lax.fori_loop(0, SB, _edge, 0)
            return 0
        lax.fori_loop(0, NSB, _sb_body, 0)

    hmask = jnp.where(lax.iota(jnp.int32, 16) < H, 1.0, 0.0)

    def _fin(i, _):
        valid = jnp.where(base + i < E, 1.0, 0.0)
        exb[i] = jnp.exp(lac[i]) * hmask * valid
        return 0
    lax.fori_loop(0, EPT, _fin, 0)
    pltpu.sync_copy(exb, ex_out.at[pl.ds(base, EPT)])

    def _scat(sb, _):
        eb = sb * SB
        for j in range(SB // 16):
            dst_idx[pl.ds(j * 16, 16)] = dst_v[pl.ds(eb + j * 16, 16)]
        pltpu.sync_copy(exb.at[pl.ds(eb, SB)], den_sh.at[dst_idx], add=True)
        return 0
    lax.fori_loop(0, NSB, _scat, 0)
    plsc.subcore_barrier()

    pltpu.sync_copy(den_sh.at[pl.ds(sid * NPT, NPT)],
                    den_out.at[cid, pl.ds(sid * NPT, NPT)])

    @pl.when(sid == NSUB - 1)
    def _wb_tail():
        pltpu.sync_copy(den_sh.at[pl.ds(NPT * NSUB, NPT_LAST - NPT)],
                        den_out.at[cid, pl.ds(NPT * NSUB, NPT_LAST - NPT)])


@functools.partial(jax.jit, static_argnums=())
def _sc_logits(xl_f, xr_f, src_p, dst_p, ea_p, wea, attf):
    mesh = plsc.VectorSubcoreMesh(core_axis_name="c", subcore_axis_name="s")
    return pl.kernel(
        _sc_logits_body,
        out_type=[
            jax.ShapeDtypeStruct((EPAD, 16), jnp.float32),
            jax.ShapeDtypeStruct((NSC, N, 16), jnp.float32),
        ],
        mesh=mesh,
        compiler_params=pltpu.CompilerParams(use_tc_tiling_on_sc=False),
        scratch_types=[
            pltpu.VMEM((EPT,), jnp.int32),
            pltpu.VMEM((EPT,), jnp.int32),
            pltpu.VMEM((EPT + 16,), jnp.float32),
            pltpu.VMEM((EPT, 16), jnp.float32),
            pltpu.VMEM((EPT, 16), jnp.float32),
            pltpu.VMEM((SB, CW), jnp.float32),
            pltpu.VMEM((SB, CW), jnp.float32),
            pltpu.VMEM((SB,), jnp.int32),
            pltpu.VMEM((SB,), jnp.int32),
            pltpu.VMEM((SB,), jnp.int32),
            pltpu.VMEM((NCH, CW), jnp.float32),
            pltpu.VMEM((NCH, CW), jnp.float32),
            pltpu.VMEM_SHARED((N, 16), jnp.float32),
            pltpu.SemaphoreType.DMA,
        ],
    )(xl_f, xr_f, src_p, dst_p, ea_p, wea, attf)


def _gatv2_layer(x, src, dst, src_p, dst_p, ea_p, Wl, bl, Wr, br, We, att, bo):
    Wcat = jnp.concatenate([Wl, Wr], axis=1)
    bcat = jnp.broadcast_to(jnp.concatenate([bl, br])[None, :], (8, 2 * HC))
    y_t = _proj_chunkmajor(x, Wcat, bcat)
    xl_f = y_t[:NCH].reshape(NCH * N, CW)
    xr_f = y_t[NCH:].reshape(NCH * N, CW)
    wea = We[0].reshape(NCH, CW)
    attf = att.reshape(HC).reshape(NCH, CW)
    ex_pad, den2 = _sc_logits(xl_f, xr_f, src_p, dst_p, ea_p, wea, attf)
    ex = ex_pad[:E, :H]
    denom = (den2[0] + den2[1])[:, :H]
    alpha = ex / (denom[dst] + 1e-16)
    xl = jnp.transpose(y_t[:NCH], (1, 0, 2)).reshape(N, H, C)
    out = jax.ops.segment_sum(xl[src] * alpha[:, :, None], dst, num_segments=N)
    return out.reshape(-1, HC) + bo


def kernel(x, edge_index, edge_attr, batch, Wl1, bl1, Wr1, br1, We1, att1, bo1,
           Wl2, bl2, Wr2, br2, We2, att2, bo2):
    src = edge_index[0]
    dst = edge_index[1]
    src_p = jnp.pad(src, (0, EPAD - E))
    dst_p = jnp.pad(dst, (0, EPAD - E))
    ea_p = jnp.pad(edge_attr[:, 0], (0, EPAD - E))
    h = _gatv2_layer(x, src, dst, src_p, dst_p, ea_p,
                     Wl1, bl1, Wr1, br1, We1, att1, bo1)
    h = _gatv2_layer(h, src, dst, src_p, dst_p, ea_p,
                     Wl2, bl2, Wr2, br2, We2, att2, bo2)
    counts = jax.ops.segment_sum(jnp.ones((N,), jnp.float32), batch,
                                 num_segments=G)
    sums = jax.ops.segment_sum(h, batch, num_segments=G)
    return sums / jnp.clip(counts, 1.0)[:, None]


# speedup vs baseline: 1.6168x; 1.6168x over previous
import functools

import jax
import jax.numpy as jnp
from jax import lax
from jax.experimental import pallas as pl
from jax.experimental.pallas import tpu as pltpu
from jax.experimental.pallas import tpu_sc as plsc

N = 10000
E = 32000
D = 2560
H = 8
C = 320
G = 16
HC = H * C
CW = 128
NCH = HC // CW
BN = 1024
BC = 512

NSC = 2
NSUB = 16
NW = NSC * NSUB
EPT = 1024
EPAD = NW * EPT
SB = 128
NSB = EPT // SB
NPT = 624
NPT_LAST = N - NPT * (NSUB - 1)


def _mm_body(x_ref, w_ref, b_ref, o_ref):
    acc = jnp.dot(x_ref[...], w_ref[...], preferred_element_type=jnp.float32)
    acc = acc + b_ref[0:1, :]
    for i in range(BC // CW):
        o_ref[i] = acc[:, i * CW:(i + 1) * CW]


def _proj_chunkmajor(x, Wcat, bcat2d):
    n_blocks = pl.cdiv(N, BN)
    c_blocks = (2 * HC) // BC
    return pl.pallas_call(
        _mm_body,
        grid=(n_blocks, c_blocks),
        in_specs=[
            pl.BlockSpec((BN, D), lambda n, c: (n, 0)),
            pl.BlockSpec((D, BC), lambda n, c: (0, c)),
            pl.BlockSpec((8, BC), lambda n, c: (0, c)),
        ],
        out_specs=pl.BlockSpec((BC // CW, BN, CW), lambda n, c: (c, n, 0)),
        out_shape=jax.ShapeDtypeStruct((2 * NCH, N, CW), jnp.float32),
    )(x, Wcat, bcat2d)


def _head_of(ch, j):
    return (ch * CW + j * 16) // C


def _onehot(h):
    return jnp.where(lax.iota(jnp.int32, 16) == h, 1.0, 0.0)


def _hsum(v):
    i = lax.iota(jnp.int32, 16)
    for k in (1, 2, 4, 8):
        v = v + v.at[jnp.bitwise_xor(i, k)].get(mode="promise_in_bounds")
    return v[0]


def _sc_logits_body(xl_f, xr_f, src_h, dst_h, ea_h, wea_h, att_h,
                    ex_out, den_out,
                    src_v, dst_v, ea_v, lac, exb,
                    rows_l, rows_r, idx_l, idx_r, dst_idx,
                    wea_v, att_v, den_sh, sem):
    cid = lax.axis_index("c")
    sid = lax.axis_index("s")
    wid = cid * NSUB + sid
    base = wid * EPT

    pltpu.sync_copy(src_h.at[pl.ds(base, EPT)], src_v)
    pltpu.sync_copy(dst_h.at[pl.ds(base, EPT)], dst_v)
    pltpu.sync_copy(ea_h.at[pl.ds(base, EPT)], ea_v.at[pl.ds(0, EPT)])
    pltpu.sync_copy(wea_h, wea_v)
    pltpu.sync_copy(att_h, att_v)

    def _zero(i, _):
        lac[i] = jnp.zeros((16,), jnp.float32)
        return 0
    lax.fori_loop(0, EPT, _zero, 0)
    pltpu.sync_copy(lac.at[pl.ds(0, NPT)], den_sh.at[pl.ds(sid * NPT, NPT)])

    @pl.when(sid == NSUB - 1)
    def _zero_tail():
        pltpu.sync_copy(lac.at[pl.ds(0, NPT_LAST - NPT)],
                        den_sh.at[pl.ds(NPT * NSUB, NPT_LAST - NPT)])
    plsc.subcore_barrier()

    for ch in range(NCH):
        wv = [wea_v[ch, pl.ds(j * 16, 16)] for j in range(8)]
        av = [att_v[ch, pl.ds(j * 16, 16)] for j in range(8)]

        def _sb_body(sb, _, ch=ch, wv=wv, av=av):
            eb = sb * SB
            for j in range(SB // 16):
                s16 = src_v[pl.ds(eb + j * 16, 16)]
                d16 = dst_v[pl.ds(eb + j * 16, 16)]
                idx_l[pl.ds(j * 16, 16)] = s16 + ch * N
                idx_r[pl.ds(j * 16, 16)] = d16 + ch * N
            pltpu.async_copy(xl_f.at[idx_l], rows_l, sem).wait()
            pltpu.async_copy(xr_f.at[idx_r], rows_r, sem).wait()

            def _edge(e, _):
                ea_s = ea_v[pl.ds(eb + e, 16)][0]
                acc = [jnp.zeros((16,), jnp.float32),
                       jnp.zeros((16,), jnp.float32)]
                h0 = _head_of(ch, 0)
                for j in range(8):
                    z = (rows_l[e, pl.ds(j * 16, 16)]
                         + rows_r[e, pl.ds(j * 16, 16)]
                         + ea_s * wv[j])
                    lr = jnp.maximum(z, 0.0) + 0.2 * jnp.minimum(z, 0.0)
                    g = _head_of(ch, j) - h0
                    acc[g] = acc[g] + lr * av[j]
                h1 = _head_of(ch, 7)
                upd = _hsum(acc[0]) * _onehot(h0)
                if h1 != h0:
                    upd = upd + _hsum(acc[1]) * _onehot(h1)
                lac[eb + e] = lac[eb + e] + upd
                return 0
            lax.fori_loop(0, SB, _edge, 0)
            return 0
        lax.fori_loop(0, NSB, _sb_body, 0)

    hmask = jnp.where(lax.iota(jnp.int32, 16) < H, 1.0, 0.0)

    def _fin(i, _):
        valid = jnp.where(base + i < E, 1.0, 0.0)
        exb[i] = jnp.exp(lac[i]) * hmask * valid
        return 0
    lax.fori_loop(0, EPT, _fin, 0)
    pltpu.sync_copy(exb, ex_out.at[pl.ds(base, EPT)])

    def _scat(sb, _):
        eb = sb * SB
        for j in range(SB // 16):
            dst_idx[pl.ds(j * 16, 16)] = dst_v[pl.ds(eb + j * 16, 16)]
        pltpu.sync_copy(exb.at[pl.ds(eb, SB)], den_sh.at[dst_idx], add=True)
        return 0
    lax.fori_loop(0, NSB, _scat, 0)
    plsc.subcore_barrier()

    pltpu.sync_copy(den_sh.at[pl.ds(sid * NPT, NPT)],
                    den_out.at[cid, pl.ds(sid * NPT, NPT)])

    @pl.when(sid == NSUB - 1)
    def _wb_tail():
        pltpu.sync_copy(den_sh.at[pl.ds(NPT * NSUB, NPT_LAST - NPT)],
                        den_out.at[cid, pl.ds(NPT * NSUB, NPT_LAST - NPT)])


@functools.partial(jax.jit, static_argnums=())
def _sc_logits(xl_f, xr_f, src_p, dst_p, ea_p, wea, attf):
    mesh = plsc.VectorSubcoreMesh(core_axis_name="c", subcore_axis_name="s")
    return pl.kernel(
        _sc_logits_body,
        out_type=[
            jax.ShapeDtypeStruct((EPAD, 16), jnp.float32),
            jax.ShapeDtypeStruct((NSC, N, 16), jnp.float32),
        ],
        mesh=mesh,
        compiler_params=pltpu.CompilerParams(use_tc_tiling_on_sc=False),
        scratch_types=[
            pltpu.VMEM((EPT,), jnp.int32),
            pltpu.VMEM((EPT,), jnp.int32),
            pltpu.VMEM((EPT + 16,), jnp.float32),
            pltpu.VMEM((EPT, 16), jnp.float32),
            pltpu.VMEM((EPT, 16), jnp.float32),
            pltpu.VMEM((SB, CW), jnp.float32),
            pltpu.VMEM((SB, CW), jnp.float32),
            pltpu.VMEM((SB,), jnp.int32),
            pltpu.VMEM((SB,), jnp.int32),
            pltpu.VMEM((SB,), jnp.int32),
            pltpu.VMEM((NCH, CW), jnp.float32),
            pltpu.VMEM((NCH, CW), jnp.float32),
            pltpu.VMEM_SHARED((N, 16), jnp.float32),
            pltpu.SemaphoreType.DMA,
        ],
    )(xl_f, xr_f, src_p, dst_p, ea_p, wea, attf)


EPC = EPAD // NSUB
NSBC = EPC // SB
WBR = 104
NH = 5000
TABR = NH + 8
SPT = 312


def _sc_scatter_body(xl_f, ex_h, den_h, src_h, dst_h, bo_h,
                     h_out,
                     src_v, dst_v, alpha_t, exs, d0b, d1b,
                     rows, idx_l, dst_idx, zb, wb, bo_v, tab_sh, sem):
    cid = lax.axis_index("c")
    sid = lax.axis_index("s")
    ebase = sid * EPC

    pltpu.sync_copy(src_h.at[pl.ds(ebase, EPC)], src_v)
    pltpu.sync_copy(dst_h.at[pl.ds(ebase, EPC)], dst_v)
    pltpu.sync_copy(bo_h, bo_v)

    def _alpha_sb(sb, _):
        eb = sb * SB
        for j in range(SB // 16):
            dst_idx[pl.ds(j * 16, 16)] = dst_v[pl.ds(eb + j * 16, 16)]
        pltpu.sync_copy(ex_h.at[pl.ds(ebase + eb, SB)], exs)
        pltpu.async_copy(den_h.at[dst_idx], d0b, sem).wait()
        for j in range(SB // 16):
            dst_idx[pl.ds(j * 16, 16)] = dst_idx[pl.ds(j * 16, 16)] + N
        pltpu.async_copy(den_h.at[dst_idx], d1b, sem).wait()

        def _arow(e, _):
            alpha_t[eb + e] = exs[e] / (d0b[e] + d1b[e] + 1e-16)
            return 0
        lax.fori_loop(0, SB, _arow, 0)
        return 0
    lax.fori_loop(0, NSBC, _alpha_sb, 0)

    def _zb(i, _):
        zb[i] = jnp.zeros((CW,), jnp.float32)
        return 0
    lax.fori_loop(0, WBR, _zb, 0)

    lanes = lax.iota(jnp.int32, 16)
    for p in range(2):
        for t in range(SPT // WBR):
            pltpu.sync_copy(zb, tab_sh.at[pl.ds(sid * SPT + t * WBR, WBR)])

        @pl.when(sid == NSUB - 1)
        def _ztail():
            pltpu.sync_copy(zb.at[pl.ds(0, TABR - SPT * NSUB)],
                            tab_sh.at[pl.ds(SPT * NSUB, TABR - SPT * NSUB)])
        plsc.subcore_barrier()

        def _chunk(i, _, p=p):
            ch = cid * (NCH // NSC) + i

            def _sb_body(sb, _):
                eb = sb * SB
                for j in range(SB // 16):
                    s16 = src_v[pl.ds(eb + j * 16, 16)]
                    idx_l[pl.ds(j * 16, 16)] = s16 + ch * N
                    rel = dst_v[pl.ds(eb + j * 16, 16)] - p * NH
                    ok = jnp.logical_and(rel >= 0, rel < NH)
                    dst_idx[pl.ds(j * 16, 16)] = jnp.where(ok, rel, NH)
                pltpu.async_copy(xl_f.at[idx_l], rows, sem).wait()

                def _edge(e, _):
                    av = alpha_t[eb + e]
                    for j in range(8):
                        hsp = jnp.broadcast_to((ch * CW + j * 16) // C, (16,))
                        a_v = av.at[hsp].get(mode="promise_in_bounds")
                        rows[e, pl.ds(j * 16, 16)] = (
                            rows[e, pl.ds(j * 16, 16)] * a_v)
                    return 0
                lax.fori_loop(0, SB, _edge, 0)
                pltpu.sync_copy(rows, tab_sh.at[dst_idx], add=True)
                return 0
            lax.fori_loop(0, NSBC, _sb_body, 0)
            plsc.subcore_barrier()

            def _wb_piece(off, nrows, ch=ch, p=p):
                pltpu.sync_copy(tab_sh.at[pl.ds(off, nrows)],
                                wb.at[pl.ds(0, nrows)])

                def _row(r, _):
                    for j in range(8):
                        wb[r, pl.ds(j * 16, 16)] = (
                            wb[r, pl.ds(j * 16, 16)]
                            + bo_v[ch, pl.ds(j * 16, 16)])
                    return 0
                lax.fori_loop(0, nrows, _row, 0)
                pltpu.sync_copy(wb.at[pl.ds(0, nrows)],
                                h_out.at[ch, pl.ds(p * NH + off, nrows)])
                pltpu.sync_copy(zb.at[pl.ds(0, nrows)],
                                tab_sh.at[pl.ds(off, nrows)])

            for t in range(SPT // WBR):
                _wb_piece(sid * SPT + t * WBR, WBR)

            @pl.when(sid == NSUB - 1)
            def _tail():
                _wb_piece(SPT * NSUB, NH - SPT * NSUB)
            plsc.subcore_barrier()
            return 0
        lax.fori_loop(0, NCH // NSC, _chunk, 0)


def _sc_scatter(xl_f, ex_pad, den2, src_p, dst_p, bo2d):
    mesh = plsc.VectorSubcoreMesh(core_axis_name="c", subcore_axis_name="s")
    return pl.kernel(
        _sc_scatter_body,
        out_type=jax.ShapeDtypeStruct((NCH, N, CW), jnp.float32),
        mesh=mesh,
        compiler_params=pltpu.CompilerParams(use_tc_tiling_on_sc=False),
        scratch_types=[
            pltpu.VMEM((EPC,), jnp.int32),
            pltpu.VMEM((EPC,), jnp.int32),
            pltpu.VMEM((EPC, 16), jnp.float32),
            pltpu.VMEM((SB, 16), jnp.float32),
            pltpu.VMEM((SB, 16), jnp.float32),
            pltpu.VMEM((SB, 16), jnp.float32),
            pltpu.VMEM((SB, CW), jnp.float32),
            pltpu.VMEM((SB,), jnp.int32),
            pltpu.VMEM((SB,), jnp.int32),
            pltpu.VMEM((WBR, CW), jnp.float32),
            pltpu.VMEM((WBR, CW), jnp.float32),
            pltpu.VMEM((NCH, CW), jnp.float32),
            pltpu.VMEM_SHARED((TABR, CW), jnp.float32),
            pltpu.SemaphoreType.DMA,
        ],
    )(xl_f, ex_pad, den2, src_p, dst_p, bo2d)


def _mm_cm_body(x_ref, w_ref, b_ref, o_ref):
    acc = jnp.dot(x_ref[0], w_ref[0], preferred_element_type=jnp.float32)
    for k in range(1, NCH):
        acc += jnp.dot(x_ref[k], w_ref[k], preferred_element_type=jnp.float32)
    acc = acc + b_ref[0:1, :]
    for i in range(BC // CW):
        o_ref[i] = acc[:, i * CW:(i + 1) * CW]


def _proj_chunkmajor_cm(x_t, W3, bcat2d):
    n_blocks = pl.cdiv(N, BN)
    c_blocks = (2 * HC) // BC
    return pl.pallas_call(
        _mm_cm_body,
        grid=(n_blocks, c_blocks),
        in_specs=[
            pl.BlockSpec((NCH, BN, CW), lambda n, c: (0, n, 0)),
            pl.BlockSpec((NCH, CW, BC), lambda n, c: (0, 0, c)),
            pl.BlockSpec((8, BC), lambda n, c: (0, c)),
        ],
        out_specs=pl.BlockSpec((BC // CW, BN, CW), lambda n, c: (c, n, 0)),
        out_shape=jax.ShapeDtypeStruct((2 * NCH, N, CW), jnp.float32),
    )(x_t, W3, bcat2d)


def _pool_body(x_ref, b_ref, o_ref):
    bat = b_ref[...]
    gids = lax.broadcasted_iota(jnp.int32, (G, N), 0)
    mask = jnp.where(gids == bat, 1.0, 0.0)
    counts = jnp.sum(mask, axis=1, keepdims=True)
    sums = jnp.dot(mask, x_ref[0], preferred_element_type=jnp.float32)
    o_ref[...] = sums / jnp.maximum(counts, 1.0)


def _pool(h_t, batch2d):
    return pl.pallas_call(
        _pool_body,
        grid=(NCH,),
        in_specs=[
            pl.BlockSpec((1, N, CW), lambda ch: (ch, 0, 0)),
            pl.BlockSpec((1, N), lambda ch: (0, 0)),
        ],
        out_specs=pl.BlockSpec((G, CW), lambda ch: (0, ch)),
        out_shape=jax.ShapeDtypeStruct((G, HC), jnp.float32),
    )(h_t, batch2d)


def _gatv2_layer(x_or_t, src_p, dst_p, ea_p, Wl, bl, Wr, br, We, att, bo,
                 chunk_major_in):
    Wcat = jnp.concatenate([Wl, Wr], axis=1)
    bcat = jnp.broadcast_to(jnp.concatenate([bl, br])[None, :], (8, 2 * HC))
    if chunk_major_in:
        y_t = _proj_chunkmajor_cm(x_or_t, Wcat.reshape(NCH, CW, 2 * HC), bcat)
    else:
        y_t = _proj_chunkmajor(x_or_t, Wcat, bcat)
    xl_f = y_t[:NCH].reshape(NCH * N, CW)
    xr_f = y_t[NCH:].reshape(NCH * N, CW)
    wea = We[0].reshape(NCH, CW)
    attf = att.reshape(HC).reshape(NCH, CW)
    ex_pad, den2 = _sc_logits(xl_f, xr_f, src_p, dst_p, ea_p, wea, attf)
    return _sc_scatter(xl_f, ex_pad, den2.reshape(NSC * N, 16), src_p,
                       dst_p, bo.reshape(NCH, CW))


def kernel(x, edge_index, edge_attr, batch, Wl1, bl1, Wr1, br1, We1, att1, bo1,
           Wl2, bl2, Wr2, br2, We2, att2, bo2):
    src = edge_index[0]
    dst = edge_index[1]
    src_p = jnp.pad(src, (0, EPAD - E))
    dst_p = jnp.pad(dst, (0, EPAD - E))
    ea_p = jnp.pad(edge_attr[:, 0], (0, EPAD - E))
    h_t = _gatv2_layer(x, src_p, dst_p, ea_p,
                       Wl1, bl1, Wr1, br1, We1, att1, bo1, False)
    h_t = _gatv2_layer(h_t, src_p, dst_p, ea_p,
                       Wl2, bl2, Wr2, br2, We2, att2, bo2, True)
    return _pool(h_t, batch.reshape(1, N))


# speedup vs baseline: 2.0951x; 1.2958x over previous
import functools

import jax
import jax.numpy as jnp
from jax import lax
from jax.experimental import pallas as pl
from jax.experimental.pallas import tpu as pltpu
from jax.experimental.pallas import tpu_sc as plsc

N = 10000
E = 32000
D = 2560
H = 8
C = 320
G = 16
HC = H * C
CW = 128
NCH = HC // CW
BN = 1024
BC = 512

NSC = 2
NSUB = 16
NW = NSC * NSUB
EPT = 1024
EPAD = NW * EPT
SB = 128
NSB = EPT // SB
NPT = 624
NPT_LAST = N - NPT * (NSUB - 1)


def _mm_body(x_ref, w_ref, b_ref, o_ref):
    acc = jnp.dot(x_ref[...], w_ref[...], preferred_element_type=jnp.float32)
    acc = acc + b_ref[0:1, :]
    for i in range(BC // CW):
        o_ref[i] = acc[:, i * CW:(i + 1) * CW]


def _proj_chunkmajor(x, Wcat, bcat2d):
    n_blocks = pl.cdiv(N, BN)
    c_blocks = (2 * HC) // BC
    return pl.pallas_call(
        _mm_body,
        grid=(n_blocks, c_blocks),
        in_specs=[
            pl.BlockSpec((BN, D), lambda n, c: (n, 0)),
            pl.BlockSpec((D, BC), lambda n, c: (0, c)),
            pl.BlockSpec((8, BC), lambda n, c: (0, c)),
        ],
        out_specs=pl.BlockSpec((BC // CW, BN, CW), lambda n, c: (c, n, 0)),
        out_shape=jax.ShapeDtypeStruct((2 * NCH, N, CW), jnp.float32),
    )(x, Wcat, bcat2d)


def _head_of(ch, j):
    return (ch * CW + j * 16) // C


def _onehot(h):
    return jnp.where(lax.iota(jnp.int32, 16) == h, 1.0, 0.0)


def _hsum(v):
    i = lax.iota(jnp.int32, 16)
    for k in (1, 2, 4, 8):
        v = v + v.at[jnp.bitwise_xor(i, k)].get(mode="promise_in_bounds")
    return v[0]


def _sc_logits_body(xl_f, xr_f, src_h, dst_h, ea_h, wea_h, att_h,
                    ex_out, den_out,
                    src_v, dst_v, ea_v, lac, exb,
                    rows_l, rows_r, rows_l2, rows_r2,
                    idx_l, idx_r, idx_l2, idx_r2, dst_idx,
                    wea_v, att_v, zden, den_sh, sem, sem2):
    cid = lax.axis_index("c")
    sid = lax.axis_index("s")
    wid = cid * NSUB + sid
    base = wid * EPT

    pltpu.sync_copy(src_h.at[pl.ds(base, EPT)], src_v)
    pltpu.sync_copy(dst_h.at[pl.ds(base, EPT)], dst_v)
    pltpu.sync_copy(ea_h.at[pl.ds(base, EPT)], ea_v.at[pl.ds(0, EPT)])
    pltpu.sync_copy(wea_h, wea_v)
    pltpu.sync_copy(att_h, att_v)

    def _zero(i, _):
        lac[i] = jnp.zeros((16,), jnp.float32)
        return 0
    lax.fori_loop(0, EPT, _zero, 0)

    def _zzd(i, _):
        zden[i] = jnp.zeros((16,), jnp.float32)
        return 0
    lax.fori_loop(0, SPT, _zzd, 0)

    rl = [rows_l, rows_l2]
    rr = [rows_r, rows_r2]
    il = [idx_l, idx_l2]
    ir = [idx_r, idx_r2]
    gsem = [sem, sem2]

    def _fill_idx(b, sb, ch):
        eb = sb * SB
        for j in range(SB // 16):
            il[b][pl.ds(j * 16, 16)] = src_v[pl.ds(eb + j * 16, 16)] + ch * N
            ir[b][pl.ds(j * 16, 16)] = dst_v[pl.ds(eb + j * 16, 16)] + ch * N

    def _start(b):
        pltpu.async_copy(xl_f.at[il[b]], rl[b], gsem[b])
        pltpu.async_copy(xr_f.at[ir[b]], rr[b], gsem[b])

    def _drain(b):
        pltpu.make_async_copy(xl_f.at[il[b]], rl[b], gsem[b]).wait()
        pltpu.make_async_copy(xr_f.at[ir[b]], rr[b], gsem[b]).wait()

    for ch in range(NCH):
        wv = [wea_v[ch, pl.ds(j * 16, 16)] for j in range(8)]
        av = [att_v[ch, pl.ds(j * 16, 16)] for j in range(8)]

        def _compute(b, sb, ch=ch, wv=wv, av=av):
            eb = sb * SB

            def _edge(e, _):
                ea_s = ea_v[pl.ds(eb + e, 16)][0]
                acc = [jnp.zeros((16,), jnp.float32),
                       jnp.zeros((16,), jnp.float32)]
                h0 = _head_of(ch, 0)
                for j in range(8):
                    z = (rl[b][e, pl.ds(j * 16, 16)]
                         + rr[b][e, pl.ds(j * 16, 16)]
                         + ea_s * wv[j])
                    lr = jnp.maximum(z, 0.0) + 0.2 * jnp.minimum(z, 0.0)
                    g = _head_of(ch, j) - h0
                    acc[g] = acc[g] + lr * av[j]
                h1 = _head_of(ch, 7)
                upd = _hsum(acc[0]) * _onehot(h0)
                if h1 != h0:
                    upd = upd + _hsum(acc[1]) * _onehot(h1)
                lac[eb + e] = lac[eb + e] + upd
                return 0
            lax.fori_loop(0, SB, _edge, 0)

        _fill_idx(0, 0, ch)
        _start(0)

        def _pair(g, _, ch=ch):
            _fill_idx(1, 2 * g + 1, ch)
            _start(1)
            _drain(0)
            _compute(0, 2 * g)

            @pl.when(g < NSB // 2 - 1)
            def _next():
                _fill_idx(0, 2 * g + 2, ch)
                _start(0)
            _drain(1)
            _compute(1, 2 * g + 1)
            return 0
        lax.fori_loop(0, NSB // 2, _pair, 0)

    hmask = jnp.where(lax.iota(jnp.int32, 16) < H, 1.0, 0.0)

    def _fin(i, _):
        valid = jnp.where(base + i < E, 1.0, 0.0)
        exb[i] = jnp.exp(lac[i]) * hmask * valid
        return 0
    lax.fori_loop(0, EPT, _fin, 0)
    pltpu.sync_copy(exb, ex_out.at[pl.ds(base, EPT)])

    for p in range(2):
        pltpu.sync_copy(zden, den_sh.at[pl.ds(sid * SPT, SPT)])

        @pl.when(sid == NSUB - 1)
        def _ztail():
            pltpu.sync_copy(zden.at[pl.ds(0, TABR - SPT * NSUB)],
                            den_sh.at[pl.ds(SPT * NSUB, TABR - SPT * NSUB)])
        plsc.subcore_barrier()

        def _scat(sb, _, p=p):
            eb = sb * SB
            for j in range(SB // 16):
                rel = dst_v[pl.ds(eb + j * 16, 16)] - p * NH
                ok = jnp.logical_and(rel >= 0, rel < NH)
                dst_idx[pl.ds(j * 16, 16)] = jnp.where(ok, rel, NH)
            pltpu.sync_copy(exb.at[pl.ds(eb, SB)], den_sh.at[dst_idx], add=True)
            return 0
        lax.fori_loop(0, NSB, _scat, 0)
        plsc.subcore_barrier()

        pltpu.sync_copy(den_sh.at[pl.ds(sid * SPT, SPT)],
                        den_out.at[cid, pl.ds(p * NH + sid * SPT, SPT)])

        @pl.when(sid == NSUB - 1)
        def _wb_tail(p=p):
            pltpu.sync_copy(den_sh.at[pl.ds(SPT * NSUB, NH - SPT * NSUB)],
                            den_out.at[cid, pl.ds(p * NH + SPT * NSUB,
                                                  NH - SPT * NSUB)])
        plsc.subcore_barrier()


@functools.partial(jax.jit, static_argnums=())
def _sc_logits(xl_f, xr_f, src_p, dst_p, ea_p, wea, attf):
    mesh = plsc.VectorSubcoreMesh(core_axis_name="c", subcore_axis_name="s")
    return pl.kernel(
        _sc_logits_body,
        out_type=[
            jax.ShapeDtypeStruct((EPAD, 16), jnp.float32),
            jax.ShapeDtypeStruct((NSC, N, 16), jnp.float32),
        ],
        mesh=mesh,
        compiler_params=pltpu.CompilerParams(use_tc_tiling_on_sc=False),
        scratch_types=[
            pltpu.VMEM((EPT,), jnp.int32),
            pltpu.VMEM((EPT,), jnp.int32),
            pltpu.VMEM((EPT + 16,), jnp.float32),
            pltpu.VMEM((EPT, 16), jnp.float32),
            pltpu.VMEM((EPT, 16), jnp.float32),
            pltpu.VMEM((SB, CW), jnp.float32),
            pltpu.VMEM((SB, CW), jnp.float32),
            pltpu.VMEM((SB, CW), jnp.float32),
            pltpu.VMEM((SB, CW), jnp.float32),
            pltpu.VMEM((SB,), jnp.int32),
            pltpu.VMEM((SB,), jnp.int32),
            pltpu.VMEM((SB,), jnp.int32),
            pltpu.VMEM((SB,), jnp.int32),
            pltpu.VMEM((SB,), jnp.int32),
            pltpu.VMEM((NCH, CW), jnp.float32),
            pltpu.VMEM((NCH, CW), jnp.float32),
            pltpu.VMEM((SPT, 16), jnp.float32),
            pltpu.VMEM_SHARED((TABR, 16), jnp.float32),
            pltpu.SemaphoreType.DMA,
            pltpu.SemaphoreType.DMA,
        ],
    )(xl_f, xr_f, src_p, dst_p, ea_p, wea, attf)


EPC = EPAD // NSUB
NSBC = EPC // SB
WBR = 104
NH = 5000
TABR = NH + 8
SPT = 312


def _sc_scatter_body(xl_f, ex_h, den_h, src_h, dst_h, bo_h,
                     h_out,
                     src_v, dst_v, alpha_t, exs, d0b, d1b,
                     rows, idx_l, dst_idx,
                     zb, wb, bo_v, tab_sh, sem):
    cid = lax.axis_index("c")
    sid = lax.axis_index("s")
    ebase = sid * EPC

    pltpu.sync_copy(src_h.at[pl.ds(ebase, EPC)], src_v)
    pltpu.sync_copy(dst_h.at[pl.ds(ebase, EPC)], dst_v)
    pltpu.sync_copy(bo_h, bo_v)

    def _alpha_sb(sb, _):
        eb = sb * SB
        for j in range(SB // 16):
            dst_idx[pl.ds(j * 16, 16)] = dst_v[pl.ds(eb + j * 16, 16)]
        pltpu.sync_copy(ex_h.at[pl.ds(ebase + eb, SB)], exs)
        pltpu.async_copy(den_h.at[dst_idx], d0b, sem).wait()
        for j in range(SB // 16):
            dst_idx[pl.ds(j * 16, 16)] = dst_idx[pl.ds(j * 16, 16)] + N
        pltpu.async_copy(den_h.at[dst_idx], d1b, sem).wait()

        def _arow(e, _):
            alpha_t[eb + e] = exs[e] / (d0b[e] + d1b[e] + 1e-16)
            return 0
        lax.fori_loop(0, SB, _arow, 0)
        return 0
    lax.fori_loop(0, NSBC, _alpha_sb, 0)

    def _zb(i, _):
        zb[i] = jnp.zeros((CW,), jnp.float32)
        return 0
    lax.fori_loop(0, WBR, _zb, 0)

    lanes = lax.iota(jnp.int32, 16)
    for p in range(2):
        for t in range(SPT // WBR):
            pltpu.sync_copy(zb, tab_sh.at[pl.ds(sid * SPT + t * WBR, WBR)])

        @pl.when(sid == NSUB - 1)
        def _ztail():
            pltpu.sync_copy(zb.at[pl.ds(0, TABR - SPT * NSUB)],
                            tab_sh.at[pl.ds(SPT * NSUB, TABR - SPT * NSUB)])
        plsc.subcore_barrier()

        def _chunk(i, _, p=p):
            ch = cid * (NCH // NSC) + i
            def _sb_body(sb, _):
                eb = sb * SB
                for j in range(SB // 16):
                    s16 = src_v[pl.ds(eb + j * 16, 16)]
                    idx_l[pl.ds(j * 16, 16)] = s16 + ch * N
                    rel = dst_v[pl.ds(eb + j * 16, 16)] - p * NH
                    ok = jnp.logical_and(rel >= 0, rel < NH)
                    dst_idx[pl.ds(j * 16, 16)] = jnp.where(ok, rel, NH)
                pltpu.async_copy(xl_f.at[idx_l], rows, sem).wait()

                def _edge(e, _):
                    av = alpha_t[eb + e]
                    for j in range(8):
                        hsp = jnp.broadcast_to((ch * CW + j * 16) // C, (16,))
                        a_v = av.at[hsp].get(mode="promise_in_bounds")
                        rows[e, pl.ds(j * 16, 16)] = (
                            rows[e, pl.ds(j * 16, 16)] * a_v)
                    return 0
                lax.fori_loop(0, SB, _edge, 0)
                pltpu.sync_copy(rows, tab_sh.at[dst_idx], add=True)
                return 0
            lax.fori_loop(0, NSBC, _sb_body, 0)
            plsc.subcore_barrier()

            def _wb_piece(off, nrows, ch=ch, p=p):
                pltpu.sync_copy(tab_sh.at[pl.ds(off, nrows)],
                                wb.at[pl.ds(0, nrows)])

                def _row(r, _):
                    for j in range(8):
                        wb[r, pl.ds(j * 16, 16)] = (
                            wb[r, pl.ds(j * 16, 16)]
                            + bo_v[ch, pl.ds(j * 16, 16)])
                    return 0
                lax.fori_loop(0, nrows, _row, 0)
                pltpu.sync_copy(wb.at[pl.ds(0, nrows)],
                                h_out.at[ch, pl.ds(p * NH + off, nrows)])
                pltpu.sync_copy(zb.at[pl.ds(0, nrows)],
                                tab_sh.at[pl.ds(off, nrows)])

            for t in range(SPT // WBR):
                _wb_piece(sid * SPT + t * WBR, WBR)

            @pl.when(sid == NSUB - 1)
            def _tail():
                _wb_piece(SPT * NSUB, NH - SPT * NSUB)
            plsc.subcore_barrier()
            return 0
        lax.fori_loop(0, NCH // NSC, _chunk, 0)


def _sc_scatter(xl_f, ex_pad, den2, src_p, dst_p, bo2d):
    mesh = plsc.VectorSubcoreMesh(core_axis_name="c", subcore_axis_name="s")
    return pl.kernel(
        _sc_scatter_body,
        out_type=jax.ShapeDtypeStruct((NCH, N, CW), jnp.float32),
        mesh=mesh,
        compiler_params=pltpu.CompilerParams(use_tc_tiling_on_sc=False),
        scratch_types=[
            pltpu.VMEM((EPC,), jnp.int32),
            pltpu.VMEM((EPC,), jnp.int32),
            pltpu.VMEM((EPC, 16), jnp.float32),
            pltpu.VMEM((SB, 16), jnp.float32),
            pltpu.VMEM((SB, 16), jnp.float32),
            pltpu.VMEM((SB, 16), jnp.float32),
            pltpu.VMEM((SB, CW), jnp.float32),
            pltpu.VMEM((SB,), jnp.int32),
            pltpu.VMEM((SB,), jnp.int32),
            pltpu.VMEM((WBR, CW), jnp.float32),
            pltpu.VMEM((WBR, CW), jnp.float32),
            pltpu.VMEM((NCH, CW), jnp.float32),
            pltpu.VMEM_SHARED((TABR, CW), jnp.float32),
            pltpu.SemaphoreType.DMA,
        ],
    )(xl_f, ex_pad, den2, src_p, dst_p, bo2d)


def _mm_cm_body(x_ref, w_ref, b_ref, o_ref):
    acc = jnp.dot(x_ref[0], w_ref[0], preferred_element_type=jnp.float32)
    for k in range(1, NCH):
        acc += jnp.dot(x_ref[k], w_ref[k], preferred_element_type=jnp.float32)
    acc = acc + b_ref[0:1, :]
    for i in range(BC // CW):
        o_ref[i] = acc[:, i * CW:(i + 1) * CW]


def _proj_chunkmajor_cm(x_t, W3, bcat2d):
    n_blocks = pl.cdiv(N, BN)
    c_blocks = (2 * HC) // BC
    return pl.pallas_call(
        _mm_cm_body,
        grid=(n_blocks, c_blocks),
        in_specs=[
            pl.BlockSpec((NCH, BN, CW), lambda n, c: (0, n, 0)),
            pl.BlockSpec((NCH, CW, BC), lambda n, c: (0, 0, c)),
            pl.BlockSpec((8, BC), lambda n, c: (0, c)),
        ],
        out_specs=pl.BlockSpec((BC // CW, BN, CW), lambda n, c: (c, n, 0)),
        out_shape=jax.ShapeDtypeStruct((2 * NCH, N, CW), jnp.float32),
    )(x_t, W3, bcat2d)


def _pool_body(x_ref, b_ref, o_ref):
    bat = b_ref[...]
    gids = lax.broadcasted_iota(jnp.int32, (G, N), 0)
    mask = jnp.where(gids == bat, 1.0, 0.0)
    counts = jnp.sum(mask, axis=1, keepdims=True)
    sums = jnp.dot(mask, x_ref[0], preferred_element_type=jnp.float32)
    o_ref[...] = sums / jnp.maximum(counts, 1.0)


def _pool(h_t, batch2d):
    return pl.pallas_call(
        _pool_body,
        grid=(NCH,),
        in_specs=[
            pl.BlockSpec((1, N, CW), lambda ch: (ch, 0, 0)),
            pl.BlockSpec((1, N), lambda ch: (0, 0)),
        ],
        out_specs=pl.BlockSpec((G, CW), lambda ch: (0, ch)),
        out_shape=jax.ShapeDtypeStruct((G, HC), jnp.float32),
    )(h_t, batch2d)


def _gatv2_layer(x_or_t, src_p, dst_p, ea_p, Wl, bl, Wr, br, We, att, bo,
                 chunk_major_in):
    Wcat = jnp.concatenate([Wl, Wr], axis=1)
    bcat = jnp.broadcast_to(jnp.concatenate([bl, br])[None, :], (8, 2 * HC))
    if chunk_major_in:
        y_t = _proj_chunkmajor_cm(x_or_t, Wcat.reshape(NCH, CW, 2 * HC), bcat)
    else:
        y_t = _proj_chunkmajor(x_or_t, Wcat, bcat)
    xl_f = y_t[:NCH].reshape(NCH * N, CW)
    xr_f = y_t[NCH:].reshape(NCH * N, CW)
    wea = We[0].reshape(NCH, CW)
    attf = att.reshape(HC).reshape(NCH, CW)
    ex_pad, den2 = _sc_logits(xl_f, xr_f, src_p, dst_p, ea_p, wea, attf)
    return _sc_scatter(xl_f, ex_pad, den2.reshape(NSC * N, 16), src_p,
                       dst_p, bo.reshape(NCH, CW))


def kernel(x, edge_index, edge_attr, batch, Wl1, bl1, Wr1, br1, We1, att1, bo1,
           Wl2, bl2, Wr2, br2, We2, att2, bo2):
    src = edge_index[0]
    dst = edge_index[1]
    src_p = jnp.pad(src, (0, EPAD - E))
    dst_p = jnp.pad(dst, (0, EPAD - E))
    ea_p = jnp.pad(edge_attr[:, 0], (0, EPAD - E))
    h_t = _gatv2_layer(x, src_p, dst_p, ea_p,
                       Wl1, bl1, Wr1, br1, We1, att1, bo1, False)
    h_t = _gatv2_layer(h_t, src_p, dst_p, ea_p,
                       Wl2, bl2, Wr2, br2, We2, att2, bo2, True)
    return _pool(h_t, batch.reshape(1, N))
